# baseline (speedup 1.0000x reference)
import functools
import math

import jax
import jax.numpy as jnp
from jax import lax
from jax.experimental import pallas as pl
from jax.experimental.pallas import tpu as pltpu, tpu_sc as plsc


def _tc_body(q_ref, k_ref, sal_ref, w_ref, idx_ref, *, kk, scale):
    scores = jax.lax.dot_general(
        q_ref[...], k_ref[...],
        dimension_numbers=(((1,), (1,)), ((), ())),
        preferred_element_type=jnp.float32,
    ) * scale + sal_ref[...]

    tile, s = scores.shape
    col = jax.lax.broadcasted_iota(jnp.int32, (tile, s), 1)

    cur = scores
    m = None
    exps = []
    idxs = []
    for _ in range(kk):
        mj = jnp.max(cur, axis=1, keepdims=True)
        ismax = cur == mj
        ij = jnp.min(jnp.where(ismax, col, s), axis=1, keepdims=True)
        e = jnp.ones_like(mj) if m is None else jnp.exp(mj - m)
        if m is None:
            m = mj
        exps.append(e)
        idxs.append(ij)
        cur = jnp.where(col == ij, -jnp.inf, cur)

    denom = functools.reduce(jnp.add, exps)
    inv = 1.0 / denom
    w_ref[...] = jnp.concatenate(exps, axis=1) * inv
    idx_ref[...] = jnp.concatenate(idxs, axis=1)


def _tc_stage(q2, K, sal2, kk, tile):
    n, d = q2.shape
    s = K.shape[0]
    body = functools.partial(_tc_body, kk=kk, scale=1.0 / math.sqrt(d))
    w, idx = pl.pallas_call(
        body,
        grid=(n // tile,),
        in_specs=[
            pl.BlockSpec((tile, d), lambda i: (i, 0)),
            pl.BlockSpec((s, d), lambda i: (0, 0)),
            pl.BlockSpec((1, s), lambda i: (0, 0)),
        ],
        out_specs=[
            pl.BlockSpec((tile, kk), lambda i: (i, 0)),
            pl.BlockSpec((tile, kk), lambda i: (i, 0)),
        ],
        out_shape=[
            jax.ShapeDtypeStruct((n, kk), jnp.float32),
            jax.ShapeDtypeStruct((n, kk), jnp.int32),
        ],
    )(q2, K, sal2)
    return w, idx


def _sc_combine(v_flat, idx, w, *, n, s, d, kk):
    NC, NS, L = 2, 16, 16
    NW = NC * NS
    per_w = n // NW
    CH = 32
    NG = CH // L
    NCH = per_w // CH
    DU = 4

    mesh = plsc.VectorSubcoreMesh(core_axis_name="c", subcore_axis_name="s")

    @functools.partial(
        pl.kernel, mesh=mesh,
        out_type=jax.ShapeDtypeStruct((n * d,), jnp.float32),
        compiler_params=pltpu.CompilerParams(needs_layout_passes=False),
        scratch_types=[
            pltpu.VMEM((s * d,), jnp.float32),
            pltpu.VMEM((CH * kk,), jnp.int32),
            pltpu.VMEM((CH * kk,), jnp.float32),
            pltpu.VMEM((CH * d,), jnp.float32),
        ],
    )
    def k(v_hbm, idx_hbm, w_hbm, out_hbm, vtab, idx_v, w_v, out_v):
        wid = lax.axis_index("s") * NC + lax.axis_index("c")
        base = wid * per_w
        pltpu.sync_copy(v_hbm, vtab)
        lanes = lax.broadcasted_iota(jnp.int32, (L,), 0)

        def chunk_body(c, carry):
            cbase = base + c * CH
            pltpu.sync_copy(idx_hbm.at[pl.ds(cbase * kk, CH * kk)], idx_v)
            pltpu.sync_copy(w_hbm.at[pl.ds(cbase * kk, CH * kk)], w_v)

            def group_body(g, carry2):
                t16 = g * L + lanes
                bases = []
                ws = []
                tk16 = t16 * kk
                for j in range(kk):
                    ij = plsc.load_gather(idx_v, [tk16 + j])
                    wj = plsc.load_gather(w_v, [tk16 + j])
                    bases.append(ij * d)
                    ws.append(wj)

                def d_body(d0, carry3):
                    for du in range(DU):
                        dd = d0 * DU + du
                        acc = jnp.zeros((L,), jnp.float32)
                        for j in range(kk):
                            g16 = plsc.load_gather(vtab, [bases[j] + dd])
                            acc = acc + ws[j] * g16
                        plsc.store_scatter(out_v, [t16 * d + dd], acc)
                    return carry3

                return lax.fori_loop(0, d // DU, d_body, carry2)

            r = lax.fori_loop(0, NG, group_body, carry)
            pltpu.sync_copy(out_v, out_hbm.at[pl.ds(cbase * d, CH * d)])
            return r

        lax.fori_loop(0, NCH, chunk_body, 0)

    return k(v_flat, idx, w)


def kernel(q, K, V, salience, topk):
    b, t, d = q.shape
    s = K.shape[0]
    kk = min(8, s)
    n = b * t
    q2 = q.reshape(n, d)
    sal2 = salience.reshape(1, s)
    w, idx = _tc_stage(q2, K, sal2, kk, 2048)
    rv = _sc_combine(V.reshape(-1), idx.reshape(-1), w.reshape(-1),
                     n=n, s=s, d=d, kk=kk).reshape(n, d)
    return rv.reshape(b, t, d), w.reshape(b, t, kk)


# speedup vs baseline: 41.7551x; 41.7551x over previous
import functools
import math

import jax
import jax.numpy as jnp
from jax.experimental import pallas as pl


def _body(q_ref, k_ref, v_ref, sal_ref, rv_ref, w_ref, *, kk, scale):
    scores = jax.lax.dot_general(
        q_ref[...], k_ref[...],
        dimension_numbers=(((1,), (1,)), ((), ())),
        preferred_element_type=jnp.float32,
    ) * scale + sal_ref[...]

    tile, s = scores.shape

    cur = scores
    m = None
    exps = []
    w_e = None
    for _ in range(kk):
        mj = jnp.max(cur, axis=1, keepdims=True)
        ismax = cur == mj
        e = jnp.ones_like(mj) if m is None else jnp.exp(mj - m)
        if m is None:
            m = mj
        exps.append(e)
        upd = jnp.where(ismax, e, 0.0)
        w_e = upd if w_e is None else w_e + upd
        cur = jnp.where(ismax, -jnp.inf, cur)

    denom = functools.reduce(jnp.add, exps)
    inv = 1.0 / denom

    w_ref[...] = jnp.concatenate(exps, axis=1) * inv
    w_dense = w_e * inv

    rv_ref[...] = jax.lax.dot_general(
        w_dense, v_ref[...],
        dimension_numbers=(((1,), (0,)), ((), ())),
        preferred_element_type=jnp.float32,
    )


def _run(q2, K, V, sal2, kk, tile, interpret=False):
    n, d = q2.shape
    s = K.shape[0]
    grid = (n // tile,)
    body = functools.partial(_body, kk=kk, scale=1.0 / math.sqrt(d))
    rv, w = pl.pallas_call(
        body,
        grid=grid,
        in_specs=[
            pl.BlockSpec((tile, d), lambda i: (i, 0)),
            pl.BlockSpec((s, d), lambda i: (0, 0)),
            pl.BlockSpec((s, d), lambda i: (0, 0)),
            pl.BlockSpec((1, s), lambda i: (0, 0)),
        ],
        out_specs=[
            pl.BlockSpec((tile, d), lambda i: (i, 0)),
            pl.BlockSpec((tile, kk), lambda i: (i, 0)),
        ],
        out_shape=[
            jax.ShapeDtypeStruct((n, d), jnp.float32),
            jax.ShapeDtypeStruct((n, kk), jnp.float32),
        ],
        interpret=interpret,
    )(q2, K, V, sal2)
    return rv, w


def kernel(q, K, V, salience, topk):
    b, t, d = q.shape
    s = K.shape[0]
    kk = min(8, s)
    n = b * t
    q2 = q.reshape(n, d)
    sal2 = salience.reshape(1, s)
    tile = 2048
    rv, w = _run(q2, K, V, sal2, kk, tile)
    return rv.reshape(b, t, d), w.reshape(b, t, kk)


# speedup vs baseline: 45.5227x; 1.0902x over previous
import functools
import math

import jax
import jax.numpy as jnp
from jax.experimental import pallas as pl


def _body(q_ref, k_ref, v_ref, sal_ref, rv_ref, w_ref, *, kk, scale):
    scores = jax.lax.dot_general(
        q_ref[...], k_ref[...],
        dimension_numbers=(((1,), (1,)), ((), ())),
        preferred_element_type=jnp.float32,
    ) * scale + sal_ref[...]

    cur = scores
    vals = []
    for _ in range(kk):
        mj = jnp.max(cur, axis=1, keepdims=True)
        vals.append(mj)
        cur = jnp.where(cur == mj, -jnp.inf, cur)

    m = vals[0]
    vals_mat = jnp.concatenate(vals, axis=1)
    e_mat = jnp.exp(vals_mat - m)
    inv = 1.0 / jnp.sum(e_mat, axis=1, keepdims=True)
    w_ref[...] = e_mat * inv
    w_dense = jnp.where(scores >= vals[-1], jnp.exp(scores - m), 0.0) * inv

    rv_ref[...] = jax.lax.dot_general(
        w_dense, v_ref[...],
        dimension_numbers=(((1,), (0,)), ((), ())),
        preferred_element_type=jnp.float32,
    )


def _run(q2, K, V, sal2, kk, tile, interpret=False):
    n, d = q2.shape
    s = K.shape[0]
    grid = (n // tile,)
    body = functools.partial(_body, kk=kk, scale=1.0 / math.sqrt(d))
    rv, w = pl.pallas_call(
        body,
        grid=grid,
        in_specs=[
            pl.BlockSpec((tile, d), lambda i: (i, 0)),
            pl.BlockSpec((s, d), lambda i: (0, 0)),
            pl.BlockSpec((s, d), lambda i: (0, 0)),
            pl.BlockSpec((1, s), lambda i: (0, 0)),
        ],
        out_specs=[
            pl.BlockSpec((tile, d), lambda i: (i, 0)),
            pl.BlockSpec((tile, kk), lambda i: (i, 0)),
        ],
        out_shape=[
            jax.ShapeDtypeStruct((n, d), jnp.float32),
            jax.ShapeDtypeStruct((n, kk), jnp.float32),
        ],
        interpret=interpret,
    )(q2, K, V, sal2)
    return rv, w


def kernel(q, K, V, salience, topk):
    b, t, d = q.shape
    s = K.shape[0]
    kk = min(8, s)
    n = b * t
    q2 = q.reshape(n, d)
    sal2 = salience.reshape(1, s)
    tile = 2048
    rv, w = _run(q2, K, V, sal2, kk, tile)
    return rv.reshape(b, t, d), w.reshape(b, t, kk)


# speedup vs baseline: 46.5384x; 1.0223x over previous
import functools
import math

import jax
import jax.numpy as jnp
from jax.experimental import pallas as pl
from jax.experimental.pallas import tpu as pltpu


def _body(q_ref, k_ref, v_ref, sal_ref, rv_ref, w_ref, *, kk, scale):
    scores = jax.lax.dot_general(
        q_ref[...], k_ref[...],
        dimension_numbers=(((1,), (1,)), ((), ())),
        preferred_element_type=jnp.float32,
    ) * scale + sal_ref[...]

    cur = scores
    vals = []
    for _ in range(kk):
        mj = jnp.max(cur, axis=1, keepdims=True)
        vals.append(mj)
        cur = jnp.where(cur == mj, -jnp.inf, cur)

    m = vals[0]
    vals_mat = jnp.concatenate(vals, axis=1)
    e_mat = jnp.exp(vals_mat - m)
    inv = 1.0 / jnp.sum(e_mat, axis=1, keepdims=True)
    w_ref[...] = e_mat * inv
    w_dense = jnp.where(scores >= vals[-1], jnp.exp(scores - m), 0.0) * inv

    rv_ref[...] = jax.lax.dot_general(
        w_dense, v_ref[...],
        dimension_numbers=(((1,), (0,)), ((), ())),
        preferred_element_type=jnp.float32,
    )


def _run(q2, K, V, sal2, kk, tile, interpret=False):
    n, d = q2.shape
    s = K.shape[0]
    grid = (n // tile,)
    body = functools.partial(_body, kk=kk, scale=1.0 / math.sqrt(d))
    rv, w = pl.pallas_call(
        body,
        grid=grid,
        in_specs=[
            pl.BlockSpec((tile, d), lambda i: (i, 0)),
            pl.BlockSpec((s, d), lambda i: (0, 0)),
            pl.BlockSpec((s, d), lambda i: (0, 0)),
            pl.BlockSpec((1, s), lambda i: (0, 0)),
        ],
        out_specs=[
            pl.BlockSpec((tile, d), lambda i: (i, 0)),
            pl.BlockSpec((tile, kk), lambda i: (i, 0)),
        ],
        out_shape=[
            jax.ShapeDtypeStruct((n, d), jnp.float32),
            jax.ShapeDtypeStruct((n, kk), jnp.float32),
        ],
        interpret=interpret,
        compiler_params=None if interpret else pltpu.CompilerParams(
            vmem_limit_bytes=100 * 1024 * 1024),
    )(q2, K, V, sal2)
    return rv, w


def kernel(q, K, V, salience, topk):
    b, t, d = q.shape
    s = K.shape[0]
    kk = min(8, s)
    n = b * t
    q2 = q.reshape(n, d)
    sal2 = salience.reshape(1, s)
    tile = 4096
    rv, w = _run(q2, K, V, sal2, kk, tile)
    return rv.reshape(b, t, d), w.reshape(b, t, kk)


# speedup vs baseline: 46.5733x; 1.0007x over previous
import functools
import math

import jax
import jax.numpy as jnp
from jax.experimental import pallas as pl
from jax.experimental.pallas import tpu as pltpu


def _body(q_ref, k_ref, v_ref, sal_ref, rv_ref, w_ref, *, kk, scale):
    scores = jax.lax.dot_general(
        q_ref[...], k_ref[...],
        dimension_numbers=(((1,), (1,)), ((), ())),
        preferred_element_type=jnp.float32,
    ) * scale + sal_ref[...]

    cur = scores
    vals = []
    for _ in range(kk):
        mj = jnp.max(cur, axis=1, keepdims=True)
        vals.append(mj)
        cur = jnp.where(cur == mj, -jnp.inf, cur)

    m = vals[0]
    vals_mat = jnp.concatenate(vals, axis=1)
    e_mat = jnp.exp(vals_mat - m)
    inv = 1.0 / jnp.sum(e_mat, axis=1, keepdims=True)
    w_ref[...] = e_mat * inv
    w_dense = jnp.where(scores >= vals[-1], jnp.exp(scores - m), 0.0) * inv

    rv_ref[...] = jax.lax.dot_general(
        w_dense, v_ref[...],
        dimension_numbers=(((1,), (0,)), ((), ())),
        preferred_element_type=jnp.float32,
    )


def _run(q2, K, V, sal2, kk, tile, interpret=False):
    n, d = q2.shape
    s = K.shape[0]
    grid = (n // tile,)
    body = functools.partial(_body, kk=kk, scale=1.0 / math.sqrt(d))
    rv, w = pl.pallas_call(
        body,
        grid=grid,
        in_specs=[
            pl.BlockSpec((tile, d), lambda i: (i, 0)),
            pl.BlockSpec((s, d), lambda i: (0, 0)),
            pl.BlockSpec((s, d), lambda i: (0, 0)),
            pl.BlockSpec((1, s), lambda i: (0, 0)),
        ],
        out_specs=[
            pl.BlockSpec((tile, d), lambda i: (i, 0)),
            pl.BlockSpec((tile, kk), lambda i: (i, 0)),
        ],
        out_shape=[
            jax.ShapeDtypeStruct((n, d), jnp.float32),
            jax.ShapeDtypeStruct((n, kk), jnp.float32),
        ],
        interpret=interpret,
        compiler_params=None if interpret else pltpu.CompilerParams(
            vmem_limit_bytes=100 * 1024 * 1024),
    )(q2, K, V, sal2)
    return rv, w


def kernel(q, K, V, salience, topk):
    b, t, d = q.shape
    s = K.shape[0]
    kk = min(8, s)
    n = b * t
    q2 = q.reshape(n, d)
    sal2 = salience.reshape(1, s)
    tile = 4096
    while n % tile:
        tile //= 2
    rv, w = _run(q2, K, V, sal2, kk, tile)
    return rv.reshape(b, t, d), w.reshape(b, t, kk)
